# Initial kernel scaffold; baseline (speedup 1.0000x reference)
import jax
import jax.numpy as jnp
from jax.experimental import pallas as pl


def kernel(x, type_id, size_id, edge_index, type_table, size_table, Ws1, Wn1, b1, Ws2, Wn2, b2, Ws3, Wn3, b3, pW1, pb1, pW2, pb2, qW1, qb1, qW2, qb2):
    raise NotImplementedError("write your pallas kernel here")


# speedup vs baseline: 12.3475x; 12.3475x over previous
import functools

import jax
import jax.numpy as jnp
from jax import lax
from jax.experimental import pallas as pl
from jax.experimental.pallas import tpu as pltpu
from jax.experimental.pallas import tpu_sc as plsc

N = 10000
E = 320000
D_FEAT = 128
MAX_TYPE = 16
MAX_SIZE = 64
HID = 128
OUT = 32
EMB = 64

NC, NS = 2, 16
NW = NC * NS
K = 128
NB = 80
E_PAD = NW * NB * K
NR = 10016
RPT = NR // NS
GPT = N // NS
DW = 8

MB = 2000
GRID = N // MB


def _sc_body_common(with_deg, g_hbm, src_hbm, dst_hbm, zero_hbm, zerod_hbm,
                    ones_hbm, out_hbm, outd_hbm,
                    src_v, dst_v, rows0, rows1, ones_v, g_sh, acc_sh, deg_sh,
                    sem_g, sem_s):
    c = lax.axis_index("c")
    s = lax.axis_index("s")
    wid = c * NS + s

    pltpu.sync_copy(g_hbm.at[pl.ds(s * GPT, GPT)], g_sh.at[pl.ds(s * GPT, GPT)])
    pltpu.sync_copy(zero_hbm, acc_sh.at[pl.ds(s * RPT, RPT)])
    if with_deg:
        pltpu.sync_copy(zerod_hbm, deg_sh.at[pl.ds(s * RPT, RPT)])
        pltpu.sync_copy(ones_hbm, ones_v)
    plsc.subcore_barrier()

    pltpu.sync_copy(src_hbm.at[wid], src_v)
    pltpu.sync_copy(dst_hbm.at[wid], dst_v)

    rows = (rows0, rows1)
    pltpu.async_copy(g_sh.at[src_v.at[0]], rows0, sem_g)

    def body(j2, carry):
        for t in range(2):
            j = j2 * 2 + t
            buf = rows[t]
            other = rows[1 - t]
            pltpu.make_async_copy(g_sh.at[src_v.at[j]], buf, sem_g).wait()

            @pl.when(j >= 1)
            def _():
                pltpu.make_async_copy(other, acc_sh.at[dst_v.at[j - 1]],
                                      sem_s).wait()

            @pl.when(j + 1 < NB)
            def _():
                pltpu.async_copy(g_sh.at[src_v.at[j + 1]], other, sem_g)

            pltpu.async_copy(buf, acc_sh.at[dst_v.at[j]], sem_s, add=True)
            if with_deg:
                pltpu.sync_copy(ones_v, deg_sh.at[dst_v.at[j]], add=True)
        return carry

    lax.fori_loop(0, NB // 2, body, 0)
    pltpu.make_async_copy(rows1, acc_sh.at[dst_v.at[NB - 1]], sem_s).wait()
    plsc.subcore_barrier()

    pltpu.sync_copy(acc_sh.at[pl.ds(s * RPT, RPT)],
                    out_hbm.at[pl.ds(c * NR + s * RPT, RPT)])
    if with_deg:
        pltpu.sync_copy(deg_sh.at[pl.ds(s * RPT, RPT)],
                        outd_hbm.at[pl.ds(c * NR + s * RPT, RPT)])


def _sc_body_deg(g_hbm, src_hbm, dst_hbm, zero_hbm, zerod_hbm, ones_hbm,
                 out_hbm, outd_hbm,
                 src_v, dst_v, rows0, rows1, ones_v, g_sh, acc_sh, deg_sh,
                 sem_g, sem_s):
    _sc_body_common(True, g_hbm, src_hbm, dst_hbm, zero_hbm, zerod_hbm,
                    ones_hbm, out_hbm, outd_hbm,
                    src_v, dst_v, rows0, rows1, ones_v, g_sh, acc_sh, deg_sh,
                    sem_g, sem_s)


def _sc_body_nodeg(g_hbm, src_hbm, dst_hbm, zero_hbm, out_hbm,
                   src_v, dst_v, rows0, rows1, g_sh, acc_sh, sem_g, sem_s):
    _sc_body_common(False, g_hbm, src_hbm, dst_hbm, zero_hbm, None, None,
                    out_hbm, None,
                    src_v, dst_v, rows0, rows1, None, g_sh, acc_sh, None,
                    sem_g, sem_s)


@functools.lru_cache(maxsize=2)
def _sc_segsum(with_deg):
    if with_deg:
        out_type = (jax.ShapeDtypeStruct((NC * NR, HID), jnp.bfloat16),
                    jax.ShapeDtypeStruct((NC * NR, DW), jnp.bfloat16))
    else:
        out_type = jax.ShapeDtypeStruct((NC * NR, HID), jnp.bfloat16)
    scratch = [
        pltpu.VMEM((NB, K), jnp.int32),
        pltpu.VMEM((NB, K), jnp.int32),
        pltpu.VMEM((K, HID), jnp.bfloat16),
        pltpu.VMEM((K, HID), jnp.bfloat16),
    ]
    if with_deg:
        scratch.append(pltpu.VMEM((K, DW), jnp.bfloat16))
    scratch.append(pltpu.VMEM_SHARED((N, HID), jnp.bfloat16))
    scratch.append(pltpu.VMEM_SHARED((NR, HID), jnp.bfloat16))
    if with_deg:
        scratch.append(pltpu.VMEM_SHARED((NR, DW), jnp.bfloat16))
    scratch.extend([pltpu.SemaphoreType.DMA, pltpu.SemaphoreType.DMA])
    return pl.kernel(
        _sc_body_deg if with_deg else _sc_body_nodeg,
        out_type=out_type,
        mesh=plsc.VectorSubcoreMesh(core_axis_name="c", subcore_axis_name="s"),
        scratch_types=scratch,
        compiler_params=pltpu.CompilerParams(use_tc_tiling_on_sc=False),
    )


def _segsum_deg(g, src3, dst3, zeros, zerosd, ones8):
    p, d = _sc_segsum(True)(g, src3, dst3, zeros, zerosd, ones8)
    return p.reshape(NC, NR, HID), d.reshape(NC, NR, DW)


def _segsum(g, src3, dst3, zeros):
    p = _sc_segsum(False)(g, src3, dst3, zeros)
    return p.reshape(NC, NR, HID)


def _tc0_body(x_ref, tid_ref, sid_ref, tt_ref, st_ref, wn_ref, ws_ref,
              g_ref, a_ref):
    xb = x_ref[...]
    oh_t = (lax.broadcasted_iota(jnp.int32, (MB, MAX_TYPE), 1) == tid_ref[...]
            ).astype(jnp.float32)
    te = jnp.dot(oh_t, tt_ref[...], preferred_element_type=jnp.float32)
    oh_s = (lax.broadcasted_iota(jnp.int32, (MB, MAX_SIZE), 1) == sid_ref[...]
            ).astype(jnp.float32)
    se = jnp.dot(oh_s, st_ref[...], preferred_element_type=jnp.float32)
    h0 = jnp.concatenate([xb, te, se], axis=1)
    g = jnp.dot(h0, wn_ref[...], preferred_element_type=jnp.float32)
    g_ref[...] = g.astype(jnp.bfloat16)
    a_ref[...] = jnp.dot(h0, ws_ref[...], preferred_element_type=jnp.float32)


def _tc0(x, tid, sid, tt, st, wn1, ws1):
    return pl.pallas_call(
        _tc0_body,
        grid=(GRID,),
        in_specs=[
            pl.BlockSpec((MB, D_FEAT), lambda i: (i, 0)),
            pl.BlockSpec((MB, 1), lambda i: (i, 0)),
            pl.BlockSpec((MB, 1), lambda i: (i, 0)),
            pl.BlockSpec((MAX_TYPE, 32), lambda i: (0, 0)),
            pl.BlockSpec((MAX_SIZE, 16), lambda i: (0, 0)),
            pl.BlockSpec((176, HID), lambda i: (0, 0)),
            pl.BlockSpec((176, HID), lambda i: (0, 0)),
        ],
        out_specs=[
            pl.BlockSpec((MB, HID), lambda i: (i, 0)),
            pl.BlockSpec((MB, HID), lambda i: (i, 0)),
        ],
        out_shape=[
            jax.ShapeDtypeStruct((N, HID), jnp.bfloat16),
            jax.ShapeDtypeStruct((N, HID), jnp.float32),
        ],
    )(x, tid, sid, tt, st, wn1, ws1)


def _tcmid_first_body(a_ref, p_ref, dg_ref, wn_ref, ws_ref, b_ref,
                      g_ref, a2_ref, dinv_ref):
    pm = p_ref[0].astype(jnp.float32) + p_ref[1].astype(jnp.float32)
    dg = (dg_ref[0, :, 0:1].astype(jnp.float32)
          + dg_ref[1, :, 0:1].astype(jnp.float32))
    dinv = 1.0 / jnp.maximum(dg, 1.0)
    dinv_ref[...] = dinv
    h = jnp.maximum(a_ref[...] + pm * dinv + b_ref[...], 0.0)
    g = jnp.dot(h, wn_ref[...], preferred_element_type=jnp.float32)
    g_ref[...] = g.astype(jnp.bfloat16)
    a2_ref[...] = jnp.dot(h, ws_ref[...], preferred_element_type=jnp.float32)


def _tcmid_next_body(a_ref, p_ref, dinv_in, wn_ref, ws_ref, b_ref,
                     g_ref, a2_ref):
    pm = p_ref[0].astype(jnp.float32) + p_ref[1].astype(jnp.float32)
    h = jnp.maximum(a_ref[...] + pm * dinv_in[...] + b_ref[...], 0.0)
    g = jnp.dot(h, wn_ref[...], preferred_element_type=jnp.float32)
    g_ref[...] = g.astype(jnp.bfloat16)
    a2_ref[...] = jnp.dot(h, ws_ref[...], preferred_element_type=jnp.float32)


_P_SPEC = pl.BlockSpec((NC, MB, HID), lambda i: (0, i, 0))
_DG_SPEC = pl.BlockSpec((NC, MB, DW), lambda i: (0, i, 0))
_A_SPEC = pl.BlockSpec((MB, HID), lambda i: (i, 0))
_GB_SPEC = pl.BlockSpec((MB, HID), lambda i: (i, 0))
_W_SPEC = pl.BlockSpec((HID, HID), lambda i: (0, 0))
_B_SPEC = pl.BlockSpec((1, HID), lambda i: (0, 0))
_DINV_SPEC = pl.BlockSpec((MB, 1), lambda i: (i, 0))


def _tcmid_first(a, p, dg, wn, ws, b):
    return pl.pallas_call(
        _tcmid_first_body,
        grid=(GRID,),
        in_specs=[_A_SPEC, _P_SPEC, _DG_SPEC, _W_SPEC, _W_SPEC, _B_SPEC],
        out_specs=[_GB_SPEC, _A_SPEC, _DINV_SPEC],
        out_shape=[
            jax.ShapeDtypeStruct((N, HID), jnp.bfloat16),
            jax.ShapeDtypeStruct((N, HID), jnp.float32),
            jax.ShapeDtypeStruct((N, 1), jnp.float32),
        ],
    )(a, p, dg, wn, ws, b)


def _tcmid_next(a, p, dinv, wn, ws, b):
    return pl.pallas_call(
        _tcmid_next_body,
        grid=(GRID,),
        in_specs=[_A_SPEC, _P_SPEC, _DINV_SPEC, _W_SPEC, _W_SPEC, _B_SPEC],
        out_specs=[_GB_SPEC, _A_SPEC],
        out_shape=[
            jax.ShapeDtypeStruct((N, HID), jnp.bfloat16),
            jax.ShapeDtypeStruct((N, HID), jnp.float32),
        ],
    )(a, p, dinv, wn, ws, b)


def _tc_final_body(a_ref, p_ref, dinv_ref, b_ref,
                   pw1_ref, pb1_ref, pw2_ref, pb2_ref,
                   qw1_ref, qb1_ref, qw2_ref, qb2_ref,
                   yhat_ref, z_ref, cs_ref):
    i = pl.program_id(0)
    pm = p_ref[0].astype(jnp.float32) + p_ref[1].astype(jnp.float32)
    h = jnp.maximum(a_ref[...] + pm * dinv_ref[...] + b_ref[...], 0.0)
    csum = jnp.sum(h, axis=0, keepdims=True)

    @pl.when(i == 0)
    def _():
        cs_ref[...] = csum

    @pl.when(i > 0)
    def _():
        cs_ref[...] = cs_ref[...] + csum

    @pl.when(i == GRID - 1)
    def _():
        hg = cs_ref[...] * (1.0 / N)
        t = jnp.maximum(
            jnp.dot(hg, pw1_ref[...], preferred_element_type=jnp.float32)
            + pb1_ref[...], 0.0)
        yhat_ref[...] = (jnp.dot(t, pw2_ref[...], preferred_element_type=jnp.float32)
                         + pb2_ref[...])
        u = jnp.maximum(
            jnp.dot(hg, qw1_ref[...], preferred_element_type=jnp.float32)
            + qb1_ref[...], 0.0)
        z = (jnp.dot(u, qw2_ref[...], preferred_element_type=jnp.float32)
             + qb2_ref[...])
        nrm = jnp.sqrt(jnp.sum(z * z, axis=1, keepdims=True))
        z_ref[...] = z / jnp.maximum(nrm, 1e-12)


def _tc_final(a, p, dinv, b, pw1, pb1, pw2, pb2, qw1, qb1, qw2, qb2):
    full = lambda shape: pl.BlockSpec(shape, lambda i: tuple(0 for _ in shape))
    return pl.pallas_call(
        _tc_final_body,
        grid=(GRID,),
        in_specs=[_A_SPEC, _P_SPEC, _DINV_SPEC, _B_SPEC,
                  full((HID, HID)), full((1, HID)), full((HID, OUT)), full((1, OUT)),
                  full((HID, EMB)), full((1, EMB)), full((EMB, EMB)), full((1, EMB))],
        out_specs=[full((1, OUT)), full((1, EMB))],
        out_shape=[
            jax.ShapeDtypeStruct((1, OUT), jnp.float32),
            jax.ShapeDtypeStruct((1, EMB), jnp.float32),
        ],
        scratch_shapes=[pltpu.VMEM((1, HID), jnp.float32)],
    )(a, p, dinv, b, pw1, pb1, pw2, pb2, qw1, qb1, qw2, qb2)


def kernel(x, type_id, size_id, edge_index, type_table, size_table,
           Ws1, Wn1, b1, Ws2, Wn2, b2, Ws3, Wn3, b3,
           pW1, pb1, pW2, pb2, qW1, qb1, qW2, qb2):
    src = edge_index[0].astype(jnp.int32)
    dst = edge_index[1].astype(jnp.int32)
    src3 = jnp.pad(src, (0, E_PAD - E)).reshape(NW, NB, K)
    dst3 = jnp.pad(dst, (0, E_PAD - E), constant_values=N).reshape(NW, NB, K)
    zeros = jnp.zeros((RPT, HID), jnp.bfloat16)
    zerosd = jnp.zeros((RPT, DW), jnp.bfloat16)
    ones8 = jnp.ones((K, DW), jnp.bfloat16)
    tid = type_id.astype(jnp.int32).reshape(N, 1)
    sid = size_id.astype(jnp.int32).reshape(N, 1)
    b1r = b1.reshape(1, HID)
    b2r = b2.reshape(1, HID)
    b3r = b3.reshape(1, HID)

    g1, a1 = _tc0(x, tid, sid, type_table, size_table, Wn1, Ws1)
    p1, dg = _segsum_deg(g1, src3, dst3, zeros, zerosd, ones8)
    g2, a2, dinv = _tcmid_first(a1, p1, dg, Wn2, Ws2, b1r)
    p2 = _segsum(g2, src3, dst3, zeros)
    g3, a3 = _tcmid_next(a2, p2, dinv, Wn3, Ws3, b2r)
    p3 = _segsum(g3, src3, dst3, zeros)
    yhat, z = _tc_final(a3, p3, dinv, b3r,
                        pW1, pb1.reshape(1, HID), pW2, pb2.reshape(1, OUT),
                        qW1, qb1.reshape(1, EMB), qW2, qb2.reshape(1, EMB))
    return (yhat, z)
